# Initial kernel scaffold; baseline (speedup 1.0000x reference)
import jax
import jax.numpy as jnp
from jax.experimental import pallas as pl


def kernel(query_embeddings, true_candidate_embeddings, candidates):
    raise NotImplementedError("write your pallas kernel here")


# speedup vs baseline: 5.1239x; 5.1239x over previous
import functools

import jax
import jax.numpy as jnp
from jax import lax
from jax.experimental import pallas as pl
from jax.experimental.pallas import tpu as pltpu

_K_TOP = 100
_KS = (1, 5, 10, 50, 100)
_CHUNK = 128

_NEG_INF = float("-inf")


def _roll_up(x, s, n):
    return jnp.concatenate([x[s:], x[:s]], axis=0)


def _cmpx(x, ii, j, asc_mask):
    n = x.shape[0]
    up = _roll_up(x, j, n)
    dn = _roll_up(x, n - j, n)
    bitj = (ii & j) == 0
    partner = jnp.where(bitj, up, dn)
    take_min = bitj == asc_mask
    return jnp.where(take_min, jnp.minimum(x, partner), jnp.maximum(x, partner))


def _sort_asc(x, ii):
    n = x.shape[0]
    k = 2
    while k <= n:
        asc_mask = (ii & k) == 0 if k < n else jnp.full_like(ii, True, dtype=bool)
        j = k // 2
        while j >= 1:
            x = _cmpx(x, ii, j, asc_mask)
            j //= 2
        k *= 2
    return x


def _merge_desc(x, ii):
    n = x.shape[0]
    desc = jnp.full_like(ii, False, dtype=bool)
    j = n // 2
    while j >= 1:
        x = _cmpx(x, ii, j, desc)
        j //= 2
    return x


def _topk_body(n_real, n_chunks, n_qblocks, q_total,
               cand_ref, qT_ref, tT_ref, y_ref, met_ref, buf_ref, pos_ref,
               met_acc_ref):
    ci = pl.program_id(1)
    qi = pl.program_id(0)
    n = _CHUNK
    ii = lax.broadcasted_iota(jnp.int32, (n, 1), 0)

    @pl.when(ci == 0)
    def _init():
        buf_ref[...] = jnp.full(buf_ref.shape, _NEG_INF, jnp.float32)
        pos_ref[...] = jnp.sum(qT_ref[...] * tT_ref[...], axis=0, keepdims=True)

    sc = jnp.dot(cand_ref[...], qT_ref[...], preferred_element_type=jnp.float32)
    valid = (ii + ci * _CHUNK) < n_real
    sc = jnp.where(valid, sc, _NEG_INF)

    sc = _sort_asc(sc, ii)
    m = jnp.maximum(buf_ref[...], sc)
    buf_ref[...] = _merge_desc(m, ii)

    @pl.when(ci == n_chunks - 1)
    def _finish():
        buf = buf_ref[...]
        pos = pos_ref[...]
        shifted = _roll_up(buf, n - 1, n)
        y_ref[...] = jnp.where(ii == 0, pos, shifted)
        top_mask = ii < _K_TOP
        nb = jnp.sum(jnp.where(jnp.logical_and(top_mask, buf > pos), 1.0, 0.0),
                     axis=0, keepdims=True)
        lane = lax.broadcasted_iota(jnp.int32, (1, 128), 1)
        part = jnp.zeros((1, 128), jnp.float32)
        for idx, kk in enumerate(_KS):
            hits = jnp.sum(jnp.where(nb < kk, 1.0, 0.0))
            part = part + jnp.where(lane == idx, hits, 0.0)

        @pl.when(qi == 0)
        def _():
            met_acc_ref[...] = part

        @pl.when(qi > 0)
        def _():
            met_acc_ref[...] = met_acc_ref[...] + part

        @pl.when(qi == n_qblocks - 1)
        def _():
            met_ref[...] = jnp.broadcast_to(
                met_acc_ref[...] * (1.0 / q_total), (8, 128))


def kernel(query_embeddings, true_candidate_embeddings, candidates):
    q_total, d = query_embeddings.shape
    n_real = candidates.shape[0]
    n_pad = (-n_real) % _CHUNK
    cand = jnp.pad(candidates, ((0, n_pad), (0, 0))) if n_pad else candidates
    n_chunks = cand.shape[0] // _CHUNK

    qb = min(512, q_total)
    n_qblocks = q_total // qb
    qT = query_embeddings.T
    tT = true_candidate_embeddings.T

    body = functools.partial(_topk_body, n_real, n_chunks, n_qblocks, q_total)
    y_t, met = pl.pallas_call(
        body,
        grid=(n_qblocks, n_chunks),
        in_specs=[
            pl.BlockSpec((_CHUNK, d), lambda qi, ci: (ci, 0)),
            pl.BlockSpec((d, qb), lambda qi, ci: (0, qi)),
            pl.BlockSpec((d, qb), lambda qi, ci: (0, qi)),
        ],
        out_specs=[
            pl.BlockSpec((_CHUNK, qb), lambda qi, ci: (0, qi)),
            pl.BlockSpec((8, 128), lambda qi, ci: (0, 0)),
        ],
        out_shape=[
            jax.ShapeDtypeStruct((_CHUNK, q_total), jnp.float32),
            jax.ShapeDtypeStruct((8, 128), jnp.float32),
        ],
        scratch_shapes=[
            pltpu.VMEM((_CHUNK, qb), jnp.float32),
            pltpu.VMEM((1, qb), jnp.float32),
            pltpu.VMEM((1, 128), jnp.float32),
        ],
    )(cand, qT, tT)
    return y_t[: _K_TOP + 1].T, met[0, :5]
